# baseline (speedup 1.0000x reference)
import functools

import jax
import jax.numpy as jnp
import numpy as np
from jax import lax
from jax.experimental import pallas as pl
from jax.experimental.pallas import tpu as pltpu
from jax.experimental.pallas import tpu_sc as plsc

NCLS = 80
CH = 84
CONF = 0.25
N_ANCH = 20000
K = 2000

NW = 16
RROWS = 160
NTOT = RROWS * 128
EPW = NTOT // NW
EGROUPS = EPW // 16
BLK = 1024
TCG = NTOT // BLK
INT_MIN = np.int32(-2**31)
KEY_CONF = np.int32(np.float32(CONF).view(np.int32))


def _tc_scores_body(x_ref, key_ref, sb_ref, b_ref):
    i = pl.program_id(0)
    x = x_ref[...]
    col = lax.broadcasted_iota(jnp.int32, (BLK, CH), 1)
    s = jnp.max(jnp.where(col >= CH - NCLS, x, -jnp.inf), axis=1)
    bs = jnp.sum(jnp.where(col < 4, x, 0.0), axis=1)
    bits = lax.bitcast_convert_type(s, jnp.int32)
    key = jnp.where(bits >= 0, bits, (~bits) ^ INT_MIN)
    key2 = key.reshape(BLK // 128, 128)
    sb2 = (s + bs).reshape(BLK // 128, 128)
    b2 = bs.reshape(BLK // 128, 128)
    ids = (i * BLK
           + lax.broadcasted_iota(jnp.int32, (BLK // 128, 128), 0) * 128
           + lax.broadcasted_iota(jnp.int32, (BLK // 128, 128), 1))
    valid = ids < N_ANCH
    key_ref[...] = jnp.where(valid, key2, INT_MIN)
    sb_ref[...] = jnp.where(valid, sb2, 0.0)
    b_ref[...] = jnp.where(valid, b2, 0.0)


def _sc_select_body(key_hbm, sb_hbm, b_hbm, out_hbm,
                    kbuf, sbbuf, bbuf, pub, allbuf, alli, pubf, allf, allfl,
                    lossbuf):
    wid = lax.axis_index("s")
    base = wid * EPW
    pltpu.sync_copy(key_hbm.at[pl.ds(base, EPW)], kbuf)
    pltpu.sync_copy(sb_hbm.at[pl.ds(base, EPW)], sbbuf)
    pltpu.sync_copy(b_hbm.at[pl.ds(base, EPW)], bbuf)
    iota = lax.iota(jnp.int32, 16)

    def global_sum_i32(local_vec):
        pub[...] = local_vec
        pltpu.sync_copy(pub, allbuf.at[pl.ds(wid * 16, 16)])
        plsc.subcore_barrier()
        pltpu.sync_copy(allbuf, alli)
        tot = jnp.zeros((16,), jnp.int32)
        for w in range(NW):
            tot = tot + alli[pl.ds(w * 16, 16)]
        plsc.subcore_barrier()
        return tot

    def count(pred_fn):
        def body(g, acc):
            v = kbuf[pl.ds(g * 16, 16)]
            return acc + pred_fn(g, v).astype(jnp.int32)
        acc = lax.fori_loop(0, EGROUPS, body, jnp.zeros((16,), jnp.int32))
        return jnp.broadcast_to(jnp.sum(acc), (16,))

    def key_round(i, p):
        t = p | (jnp.int32(1) << (31 - i))
        ts = t ^ INT_MIN
        tot = global_sum_i32(count(lambda g, v: v >= ts))
        return jnp.where(tot >= K, t, p)

    p = lax.fori_loop(0, 32, key_round, jnp.zeros((16,), jnp.int32))
    ts = p ^ INT_MIN

    cnt_gt = global_sum_i32(count(lambda g, v: v > ts))
    r = K - cnt_gt

    def idx_round(i, q):
        t2 = q | (jnp.int32(1) << (14 - i))

        def pred(g, v):
            ivec = base + g * 16 + iota
            return (v == ts) & (ivec < t2)

        tot = global_sum_i32(count(pred))
        return jnp.where(tot < r, t2, q)

    m = lax.fori_loop(0, 15, idx_round, jnp.zeros((16,), jnp.int32))

    def sums_body(g, carry):
        a_gt, a_tb, a_cf = carry
        v = kbuf[pl.ds(g * 16, 16)]
        sbv = sbbuf[pl.ds(g * 16, 16)]
        bv = bbuf[pl.ds(g * 16, 16)]
        ivec = base + g * 16 + iota
        a_gt = a_gt + jnp.where(v > ts, sbv, 0.0)
        a_tb = a_tb + jnp.where((v == ts) & (ivec <= m), bv, 0.0)
        a_cf = a_cf + jnp.where(v >= KEY_CONF, sbv, 0.0)
        return a_gt, a_tb, a_cf

    z = jnp.zeros((16,), jnp.float32)
    a_gt, a_tb, a_cf = lax.fori_loop(0, EGROUPS, sums_body, (z, z, z))
    pubf[pl.ds(0, 16)] = jnp.broadcast_to(jnp.sum(a_gt), (16,))
    pubf[pl.ds(16, 16)] = jnp.broadcast_to(jnp.sum(a_tb), (16,))
    pubf[pl.ds(32, 16)] = jnp.broadcast_to(jnp.sum(a_cf), (16,))
    pltpu.sync_copy(pubf, allf.at[pl.ds(wid * 48, 48)])
    plsc.subcore_barrier()

    @pl.when(wid == 0)
    def _():
        pltpu.sync_copy(allf, allfl)
        tg = jnp.zeros((16,), jnp.float32)
        tt = jnp.zeros((16,), jnp.float32)
        tc = jnp.zeros((16,), jnp.float32)
        for w in range(NW):
            tg = tg + allfl[pl.ds(w * 48, 16)]
            tt = tt + allfl[pl.ds(w * 48 + 16, 16)]
            tc = tc + allfl[pl.ds(w * 48 + 32, 16)]
        beta = jnp.where(p < 0, p ^ INT_MIN, ~p)
        t_f = plsc.bitcast(beta, jnp.float32)
        loss_a = tg + r.astype(jnp.float32) * t_f + tt
        loss = jnp.where(t_f >= CONF, loss_a, tc)
        lossbuf[...] = loss
        pltpu.sync_copy(lossbuf.at[pl.ds(0, 8)], out_hbm.at[pl.ds(0, 8)])


@functools.cache
def _sc_select():
    return pl.kernel(
        _sc_select_body,
        out_type=jax.ShapeDtypeStruct((8,), jnp.float32),
        mesh=plsc.VectorSubcoreMesh(core_axis_name="c", subcore_axis_name="s",
                                    num_cores=1, num_subcores=16),
        scratch_types=[pltpu.VMEM((EPW,), jnp.int32),
                       pltpu.VMEM((EPW,), jnp.float32),
                       pltpu.VMEM((EPW,), jnp.float32),
                       pltpu.VMEM((16,), jnp.int32),
                       pltpu.VMEM_SHARED((NW * 16,), jnp.int32),
                       pltpu.VMEM((NW * 16,), jnp.int32),
                       pltpu.VMEM((48,), jnp.float32),
                       pltpu.VMEM_SHARED((NW * 48,), jnp.float32),
                       pltpu.VMEM((NW * 48,), jnp.float32),
                       pltpu.VMEM((16,), jnp.float32)],
        compiler_params=pltpu.CompilerParams(needs_layout_passes=False),
    )


def kernel(data):
    keys, sb, b = pl.pallas_call(
        _tc_scores_body,
        grid=(TCG,),
        in_specs=[pl.BlockSpec((BLK, CH), lambda i: (i, 0))],
        out_specs=[pl.BlockSpec((BLK // 128, 128), lambda i: (i, 0))] * 3,
        out_shape=[jax.ShapeDtypeStruct((RROWS, 128), jnp.int32),
                   jax.ShapeDtypeStruct((RROWS, 128), jnp.float32),
                   jax.ShapeDtypeStruct((RROWS, 128), jnp.float32)],
    )(data[0])
    loss8 = _sc_select()(keys.reshape(NTOT), sb.reshape(NTOT),
                         b.reshape(NTOT))
    return loss8[0]


# speedup vs baseline: 1.1017x; 1.1017x over previous
import functools

import jax
import jax.numpy as jnp
import numpy as np
from jax import lax
from jax.experimental import pallas as pl
from jax.experimental.pallas import tpu as pltpu
from jax.experimental.pallas import tpu_sc as plsc

NCLS = 80
CH = 84
CONF = 0.25
N_ANCH = 20000
K = 2000

NW = 16
RROWS = 160
NTOT = RROWS * 128
EPW = NTOT // NW
EGROUPS = EPW // 16
BLK = 1024
TCG = NTOT // BLK
INT_MIN = np.int32(-2**31)
KEY_CONF = np.int32(np.float32(CONF).view(np.int32))


def _tc_scores_body(x_ref, key_ref, sb_ref, b_ref):
    i = pl.program_id(0)
    x = x_ref[...]
    col = lax.broadcasted_iota(jnp.int32, (BLK, CH), 1)
    s = jnp.max(jnp.where(col >= CH - NCLS, x, -jnp.inf), axis=1)
    bs = jnp.sum(jnp.where(col < 4, x, 0.0), axis=1)
    bits = lax.bitcast_convert_type(s, jnp.int32)
    key = jnp.where(bits >= 0, bits, (~bits) ^ INT_MIN)
    key2 = key.reshape(BLK // 128, 128)
    sb2 = (s + bs).reshape(BLK // 128, 128)
    b2 = bs.reshape(BLK // 128, 128)
    ids = (i * BLK
           + lax.broadcasted_iota(jnp.int32, (BLK // 128, 128), 0) * 128
           + lax.broadcasted_iota(jnp.int32, (BLK // 128, 128), 1))
    valid = ids < N_ANCH
    key_ref[...] = jnp.where(valid, key2, INT_MIN)
    sb_ref[...] = jnp.where(valid, sb2, 0.0)
    b_ref[...] = jnp.where(valid, b2, 0.0)


def _sc_select_body(key_hbm, sb_hbm, b_hbm, out_hbm,
                    kbuf, sbbuf, bbuf, pub, allbuf, alli, pubf, allf, allfl,
                    lossbuf):
    wid = lax.axis_index("s")
    base = wid * EPW
    pltpu.sync_copy(key_hbm.at[pl.ds(base, EPW)], kbuf)
    pltpu.sync_copy(sb_hbm.at[pl.ds(base, EPW)], sbbuf)
    pltpu.sync_copy(b_hbm.at[pl.ds(base, EPW)], bbuf)
    iota = lax.iota(jnp.int32, 16)

    def global_sum_i32(local_vec):
        pub[...] = local_vec
        pltpu.sync_copy(pub, allbuf.at[pl.ds(wid * 16, 16)])
        plsc.subcore_barrier()
        pltpu.sync_copy(allbuf, alli)
        tot = jnp.zeros((16,), jnp.int32)
        for w in range(NW):
            tot = tot + alli[pl.ds(w * 16, 16)]
        plsc.subcore_barrier()
        return tot

    def count(pred_fn):
        accs = [jnp.zeros((16,), jnp.int32) for _ in range(4)]
        for g in range(EGROUPS):
            v = kbuf[pl.ds(g * 16, 16)]
            accs[g % 4] = accs[g % 4] + pred_fn(g, v).astype(jnp.int32)
        acc = (accs[0] + accs[1]) + (accs[2] + accs[3])
        return jnp.broadcast_to(jnp.sum(acc), (16,))

    def key_round(i, p):
        t = p | (jnp.int32(1) << (31 - i))
        ts = t ^ INT_MIN
        tot = global_sum_i32(count(lambda g, v: v >= ts))
        return jnp.where(tot >= K, t, p)

    p = lax.fori_loop(0, 32, key_round, jnp.zeros((16,), jnp.int32))
    ts = p ^ INT_MIN

    cnt_gt = global_sum_i32(count(lambda g, v: v > ts))
    r = K - cnt_gt

    def idx_round(i, q):
        t2 = q | (jnp.int32(1) << (14 - i))

        def pred(g, v):
            ivec = base + g * 16 + iota
            return (v == ts) & (ivec < t2)

        tot = global_sum_i32(count(pred))
        return jnp.where(tot < r, t2, q)

    m = lax.fori_loop(0, 15, idx_round, jnp.zeros((16,), jnp.int32))

    z = jnp.zeros((16,), jnp.float32)
    a_gt, a_tb, a_cf = z, z, z
    for g in range(EGROUPS):
        v = kbuf[pl.ds(g * 16, 16)]
        sbv = sbbuf[pl.ds(g * 16, 16)]
        bv = bbuf[pl.ds(g * 16, 16)]
        ivec = base + g * 16 + iota
        a_gt = a_gt + jnp.where(v > ts, sbv, 0.0)
        a_tb = a_tb + jnp.where((v == ts) & (ivec <= m), bv, 0.0)
        a_cf = a_cf + jnp.where(v >= KEY_CONF, sbv, 0.0)
    pubf[pl.ds(0, 16)] = jnp.broadcast_to(jnp.sum(a_gt), (16,))
    pubf[pl.ds(16, 16)] = jnp.broadcast_to(jnp.sum(a_tb), (16,))
    pubf[pl.ds(32, 16)] = jnp.broadcast_to(jnp.sum(a_cf), (16,))
    pltpu.sync_copy(pubf, allf.at[pl.ds(wid * 48, 48)])
    plsc.subcore_barrier()

    @pl.when(wid == 0)
    def _():
        pltpu.sync_copy(allf, allfl)
        tg = jnp.zeros((16,), jnp.float32)
        tt = jnp.zeros((16,), jnp.float32)
        tc = jnp.zeros((16,), jnp.float32)
        for w in range(NW):
            tg = tg + allfl[pl.ds(w * 48, 16)]
            tt = tt + allfl[pl.ds(w * 48 + 16, 16)]
            tc = tc + allfl[pl.ds(w * 48 + 32, 16)]
        beta = jnp.where(p < 0, p ^ INT_MIN, ~p)
        t_f = plsc.bitcast(beta, jnp.float32)
        loss_a = tg + r.astype(jnp.float32) * t_f + tt
        loss = jnp.where(t_f >= CONF, loss_a, tc)
        lossbuf[...] = loss
        pltpu.sync_copy(lossbuf.at[pl.ds(0, 8)], out_hbm.at[pl.ds(0, 8)])


@functools.cache
def _sc_select():
    return pl.kernel(
        _sc_select_body,
        out_type=jax.ShapeDtypeStruct((8,), jnp.float32),
        mesh=plsc.VectorSubcoreMesh(core_axis_name="c", subcore_axis_name="s",
                                    num_cores=1, num_subcores=16),
        scratch_types=[pltpu.VMEM((EPW,), jnp.int32),
                       pltpu.VMEM((EPW,), jnp.float32),
                       pltpu.VMEM((EPW,), jnp.float32),
                       pltpu.VMEM((16,), jnp.int32),
                       pltpu.VMEM_SHARED((NW * 16,), jnp.int32),
                       pltpu.VMEM((NW * 16,), jnp.int32),
                       pltpu.VMEM((48,), jnp.float32),
                       pltpu.VMEM_SHARED((NW * 48,), jnp.float32),
                       pltpu.VMEM((NW * 48,), jnp.float32),
                       pltpu.VMEM((16,), jnp.float32)],
        compiler_params=pltpu.CompilerParams(needs_layout_passes=False),
    )


def kernel(data):
    keys, sb, b = pl.pallas_call(
        _tc_scores_body,
        grid=(TCG,),
        in_specs=[pl.BlockSpec((BLK, CH), lambda i: (i, 0))],
        out_specs=[pl.BlockSpec((BLK // 128, 128), lambda i: (i, 0))] * 3,
        out_shape=[jax.ShapeDtypeStruct((RROWS, 128), jnp.int32),
                   jax.ShapeDtypeStruct((RROWS, 128), jnp.float32),
                   jax.ShapeDtypeStruct((RROWS, 128), jnp.float32)],
    )(data[0])
    loss8 = _sc_select()(keys.reshape(NTOT), sb.reshape(NTOT),
                         b.reshape(NTOT))
    return loss8[0]


# speedup vs baseline: 1.1822x; 1.0731x over previous
import functools

import jax
import jax.numpy as jnp
import numpy as np
from jax import lax
from jax.experimental import pallas as pl
from jax.experimental.pallas import tpu as pltpu
from jax.experimental.pallas import tpu_sc as plsc

NCLS = 80
CH = 84
CONF = 0.25
N_ANCH = 20000
K = 2000

NW = 32
APW = N_ANCH // NW
ROW = 640
GROUPS = ROW // 16
WORDS = APW * CH + 4
INT_MIN = np.int32(-2**31)


def _sc_scores_body(data_hbm, sc_out, bx_out, buf, srow, brow):
    wid = lax.axis_index("s") * 2 + lax.axis_index("c")
    gstart = wid * (APW * CH)
    astart = (gstart // 8) * 8
    off = gstart - astart
    pltpu.sync_copy(data_hbm.at[pl.ds(astart, WORDS)], buf)
    iota = lax.iota(jnp.int32, 16)

    def group(g, carry):
        anchors = g * 16 + iota
        valid = anchors < APW
        base = off + jnp.minimum(anchors, APW - 1) * CH
        accs = [jnp.full((16,), -jnp.inf, jnp.float32) for _ in range(4)]
        for c in range(4, CH):
            v = plsc.load_gather(buf, [base + c])
            accs[c % 4] = jnp.maximum(accs[c % 4], v)
        s = jnp.maximum(jnp.maximum(accs[0], accs[1]),
                        jnp.maximum(accs[2], accs[3]))
        bsum = plsc.load_gather(buf, [base])
        for c in range(1, 4):
            bsum = bsum + plsc.load_gather(buf, [base + c])
        srow[pl.ds(g * 16, 16)] = jnp.where(valid, s, -jnp.inf)
        brow[pl.ds(g * 16, 16)] = jnp.where(valid, bsum, 0.0)
        return carry

    lax.fori_loop(0, GROUPS, group, jnp.int32(0))
    pltpu.sync_copy(srow, sc_out.at[wid])
    pltpu.sync_copy(brow, bx_out.at[wid])


@functools.cache
def _sc_scores():
    return pl.kernel(
        _sc_scores_body,
        out_type=(jax.ShapeDtypeStruct((NW, ROW), jnp.float32),
                  jax.ShapeDtypeStruct((NW, ROW), jnp.float32)),
        mesh=plsc.VectorSubcoreMesh(core_axis_name="c", subcore_axis_name="s",
                                    num_cores=2, num_subcores=16),
        scratch_types=[pltpu.VMEM((WORDS,), jnp.float32),
                       pltpu.VMEM((ROW,), jnp.float32),
                       pltpu.VMEM((ROW,), jnp.float32)],
        compiler_params=pltpu.CompilerParams(needs_layout_passes=False),
    )


def _tc_select_body(s_ref, b_ref, out_ref):
    s = s_ref[...]
    b = b_ref[...]
    col = lax.broadcasted_iota(jnp.int32, (NW, ROW), 1)
    row = lax.broadcasted_iota(jnp.int32, (NW, ROW), 0)
    valid = col < APW
    bits = lax.bitcast_convert_type(s, jnp.int32)
    key = jnp.where(bits >= 0, bits, (~bits) ^ INT_MIN)
    key = jnp.where(valid, key, INT_MIN)
    idx = jnp.where(valid, row * APW + col, jnp.int32(1 << 30))

    def key_bit(i, p):
        t = p | (jnp.int32(1) << (31 - i))
        cnt = jnp.sum((key >= (t ^ INT_MIN)).astype(jnp.int32))
        return jnp.where(cnt >= K, t, p)

    p_u = lax.fori_loop(0, 32, key_bit, jnp.int32(0))
    t_s = p_u ^ INT_MIN
    cnt_gt = jnp.sum((key > t_s).astype(jnp.int32))
    r = K - cnt_gt
    tie = key == t_s

    def idx_bit(i, q):
        t = q | (jnp.int32(1) << (14 - i))
        c = jnp.sum((tie & (idx < t)).astype(jnp.int32))
        return jnp.where(c < r, t, q)

    m = lax.fori_loop(0, 15, idx_bit, jnp.int32(0))

    beta = jnp.where(p_u < 0, p_u ^ INT_MIN, ~p_u)
    t_f = lax.bitcast_convert_type(beta, jnp.float32)

    sum_gt = jnp.sum(jnp.where(key > t_s, s + b, 0.0))
    sum_tie_b = jnp.sum(jnp.where(tie & (idx <= m), b, 0.0))
    loss_a = sum_gt + r.astype(jnp.float32) * t_f + sum_tie_b
    loss_b = jnp.sum(jnp.where(valid & (s >= CONF), s + b, 0.0))
    out_ref[0, 0] = jnp.where(t_f >= CONF, loss_a, loss_b)


def kernel(data):
    flat = data.reshape(N_ANCH * CH)
    scores, boxsum = _sc_scores()(flat)
    out = pl.pallas_call(
        _tc_select_body,
        out_shape=jax.ShapeDtypeStruct((1, 1), jnp.float32),
        out_specs=pl.BlockSpec(memory_space=pltpu.SMEM),
    )(scores, boxsum)
    return out.reshape(())
